# baseline (speedup 1.0000x reference)
import functools

import jax
import jax.numpy as jnp
from jax import lax
from jax.experimental import pallas as pl
from jax.experimental.pallas import tpu as pltpu
from jax.experimental.pallas import tpu_sc as plsc

N = 8388608
T = 8
H = 26
W = 40
BINS = 2 * T * H * W


TC_COLS = 1024
TC_ROWS = N // 2 // TC_COLS
TC_BLOCK_ROWS = 512
TC_GRID = TC_ROWS // TC_BLOCK_ROWS


def _quant_body(xl, yl, tl, pl_, xh, yh, th, ph, o_ref):
    def flat(xr, yr, tr, pr):
        xi = xr[...] >> 3
        yi = jnp.minimum(yr[...] >> 3, H - 1)
        ti = (tr[...] * jnp.float32(T)).astype(jnp.int32)
        return ((pr[...] << 3) + ti) * (H * W) + yi * W + xi

    lo = flat(xl, yl, tl, pl_)
    hi = flat(xh, yh, th, ph)
    o_ref[...] = (hi << 16) | lo


def _quantize(x, y, t, p):
    lo_spec = pl.BlockSpec((TC_BLOCK_ROWS, TC_COLS), lambda i: (i, 0))
    hi_spec = pl.BlockSpec((TC_BLOCK_ROWS, TC_COLS), lambda i: (i + TC_GRID, 0))
    args = []
    for a in (x, y, t, p):
        args.append(a.reshape(2 * TC_ROWS, TC_COLS))
    packed = pl.pallas_call(
        _quant_body,
        grid=(TC_GRID,),
        in_specs=[lo_spec] * 4 + [hi_spec] * 4,
        out_specs=pl.BlockSpec((TC_BLOCK_ROWS, TC_COLS), lambda i: (i, 0)),
        out_shape=jax.ShapeDtypeStruct((TC_ROWS, TC_COLS), jnp.int32),
    )(*args, *args)
    return packed.reshape(N // 2)


NC = 2
NS = 16
NW = NC * NS
PER_W = N // 2 // NW
C = 32768
N_CHUNKS = PER_W // C
L = 16
VPC = C // L
SLICE = BINS // NS


def _hist_body(f_hbm, out_hbm, fa, fb, histv, acc, tmp, slots, sem_a, sem_b):
    cid = lax.axis_index("c")
    sid = lax.axis_index("s")
    wid = sid * NC + cid
    ev_base = wid * PER_W

    zero16 = jnp.zeros((L,), dtype=jnp.float32)
    one16 = jnp.full((L,), 1.0, dtype=jnp.float32)

    def fill_zero(i, carry):
        histv[pl.ds(i * L, L)] = zero16
        return carry

    lax.fori_loop(0, BINS // L, fill_zero, 0)

    def accumulate(fr):
        @plsc.parallel_loop(0, VPC, unroll=8)
        def vec_body(i):
            w = fr[pl.ds(i * L, L)]
            lo = w & jnp.int32(0xFFFF)
            hi = w >> 16
            plsc.addupdate_scatter(histv, [lo], one16)
            plsc.addupdate_scatter(histv, [hi], one16)

    def start_load(base, buf, sem):
        pltpu.async_copy(f_hbm.at[pl.ds(base, C)], buf, sem)

    def wait_load(buf, sem):
        pltpu.make_async_copy(f_hbm.at[pl.ds(0, C)], buf, sem).wait()

    start_load(ev_base, fa, sem_a)

    def chunk_pair(jj, carry):
        start_load(ev_base + (2 * jj + 1) * C, fb, sem_b)
        wait_load(fa, sem_a)
        accumulate(fa)

        @pl.when(jj + 1 < N_CHUNKS // 2)
        def _():
            start_load(ev_base + (2 * jj + 2) * C, fa, sem_a)

        wait_load(fb, sem_b)
        accumulate(fb)
        return carry

    lax.fori_loop(0, N_CHUNKS // 2, chunk_pair, 0)

    pltpu.sync_copy(histv, slots.at[pl.ds(sid * BINS, BINS)])
    plsc.subcore_barrier()

    off = sid * SLICE
    pltpu.sync_copy(slots.at[pl.ds(off, SLICE)], acc)

    def red_body(k, carry):
        pltpu.sync_copy(slots.at[pl.ds(k * BINS + off, SLICE)], tmp)

        def add_body(i, carry2):
            s = pl.ds(i * L, L)
            acc[s] = acc[s] + tmp[s]
            return carry2

        lax.fori_loop(0, SLICE // L, add_body, 0)
        return carry

    lax.fori_loop(1, NS, red_body, 0)
    pltpu.sync_copy(acc, out_hbm.at[pl.ds(cid * BINS + off, SLICE)])


def _scatter(packed):
    mesh = plsc.VectorSubcoreMesh(
        core_axis_name="c", subcore_axis_name="s",
        num_cores=NC, num_subcores=NS,
    )
    return pl.kernel(
        _hist_body,
        out_type=jax.ShapeDtypeStruct((NC * BINS,), jnp.float32),
        mesh=mesh,
        compiler_params=pltpu.CompilerParams(needs_layout_passes=False),
        scratch_types=[
            pltpu.VMEM((C,), jnp.int32),
            pltpu.VMEM((C,), jnp.int32),
            pltpu.VMEM((BINS,), jnp.float32),
            pltpu.VMEM((SLICE,), jnp.float32),
            pltpu.VMEM((SLICE,), jnp.float32),
            pltpu.VMEM_SHARED((NS * BINS,), jnp.float32),
            pltpu.SemaphoreType.DMA,
            pltpu.SemaphoreType.DMA,
        ],
    )(packed)


@jax.jit
def _voxel_hist(x, y, t, p):
    packed = _quantize(x, y, t, p)
    partials = _scatter(packed)
    return partials.reshape(NC, BINS).sum(axis=0).reshape(2 * T, H, W)


def kernel(x, y, t, p):
    return _voxel_hist(x, y, t, p)


# speedup vs baseline: 2.4665x; 2.4665x over previous
import functools

import jax
import jax.numpy as jnp
from jax import lax
from jax.experimental import pallas as pl
from jax.experimental.pallas import tpu as pltpu
from jax.experimental.pallas import tpu_sc as plsc

N = 8388608
T = 8
H = 26
W = 40
BINS = 2 * T * H * W


TC_BLOCK = 262144
TC_GRID = N // 2 // TC_BLOCK


def _quant_body(xl, yl, tl, pl_, xh, yh, th, ph, o_ref):
    def flat(xr, yr, tr, pr):
        xi = xr[...] >> 3
        yi = jnp.minimum(yr[...] >> 3, H - 1)
        ti = (tr[...] * jnp.float32(T)).astype(jnp.int32)
        return ((pr[...] << 3) + ti) * (H * W) + yi * W + xi

    lo = flat(xl, yl, tl, pl_)
    hi = flat(xh, yh, th, ph)
    o_ref[...] = (hi << 16) | lo


def _quantize(x, y, t, p):
    lo_spec = pl.BlockSpec((TC_BLOCK,), lambda i: (i,))
    hi_spec = pl.BlockSpec((TC_BLOCK,), lambda i: (i + TC_GRID,))
    packed = pl.pallas_call(
        _quant_body,
        grid=(TC_GRID,),
        in_specs=[lo_spec] * 4 + [hi_spec] * 4,
        out_specs=pl.BlockSpec((TC_BLOCK,), lambda i: (i,)),
        out_shape=jax.ShapeDtypeStruct((N // 2,), jnp.int32),
    )(x, y, t, p, x, y, t, p)
    return packed


NC = 2
NS = 16
NW = NC * NS
PER_W = N // 2 // NW
C = 32768
N_CHUNKS = PER_W // C
L = 16
VPC = C // L
SLICE = BINS // NS


def _hist_body(f_hbm, out_hbm, fa, fb, histv, acc, tmp, slots, sem_a, sem_b):
    cid = lax.axis_index("c")
    sid = lax.axis_index("s")
    wid = sid * NC + cid
    ev_base = wid * PER_W

    zero16 = jnp.zeros((L,), dtype=jnp.float32)
    one16 = jnp.full((L,), 1.0, dtype=jnp.float32)

    def fill_zero(i, carry):
        histv[pl.ds(i * L, L)] = zero16
        return carry

    lax.fori_loop(0, BINS // L, fill_zero, 0)

    def accumulate(fr):
        @plsc.parallel_loop(0, VPC, unroll=8)
        def vec_body(i):
            w = fr[pl.ds(i * L, L)]
            lo = w & jnp.int32(0xFFFF)
            hi = w >> 16
            plsc.addupdate_scatter(histv, [lo], one16)
            plsc.addupdate_scatter(histv, [hi], one16)

    def start_load(base, buf, sem):
        pltpu.async_copy(f_hbm.at[pl.ds(base, C)], buf, sem)

    def wait_load(buf, sem):
        pltpu.make_async_copy(f_hbm.at[pl.ds(0, C)], buf, sem).wait()

    start_load(ev_base, fa, sem_a)

    def chunk_pair(jj, carry):
        start_load(ev_base + (2 * jj + 1) * C, fb, sem_b)
        wait_load(fa, sem_a)
        accumulate(fa)

        @pl.when(jj + 1 < N_CHUNKS // 2)
        def _():
            start_load(ev_base + (2 * jj + 2) * C, fa, sem_a)

        wait_load(fb, sem_b)
        accumulate(fb)
        return carry

    lax.fori_loop(0, N_CHUNKS // 2, chunk_pair, 0)

    pltpu.sync_copy(histv, slots.at[pl.ds(sid * BINS, BINS)])
    plsc.subcore_barrier()

    off = sid * SLICE
    pltpu.sync_copy(slots.at[pl.ds(off, SLICE)], acc)

    def red_body(k, carry):
        pltpu.sync_copy(slots.at[pl.ds(k * BINS + off, SLICE)], tmp)

        def add_body(i, carry2):
            s = pl.ds(i * L, L)
            acc[s] = acc[s] + tmp[s]
            return carry2

        lax.fori_loop(0, SLICE // L, add_body, 0)
        return carry

    lax.fori_loop(1, NS, red_body, 0)
    pltpu.sync_copy(acc, out_hbm.at[pl.ds(cid * BINS + off, SLICE)])


def _scatter(packed):
    mesh = plsc.VectorSubcoreMesh(
        core_axis_name="c", subcore_axis_name="s",
        num_cores=NC, num_subcores=NS,
    )
    return pl.kernel(
        _hist_body,
        out_type=jax.ShapeDtypeStruct((NC * BINS,), jnp.float32),
        mesh=mesh,
        compiler_params=pltpu.CompilerParams(needs_layout_passes=False),
        scratch_types=[
            pltpu.VMEM((C,), jnp.int32),
            pltpu.VMEM((C,), jnp.int32),
            pltpu.VMEM((BINS,), jnp.float32),
            pltpu.VMEM((SLICE,), jnp.float32),
            pltpu.VMEM((SLICE,), jnp.float32),
            pltpu.VMEM_SHARED((NS * BINS,), jnp.float32),
            pltpu.SemaphoreType.DMA,
            pltpu.SemaphoreType.DMA,
        ],
    )(packed)


@jax.jit
def _voxel_hist(x, y, t, p):
    packed = _quantize(x, y, t, p)
    partials = _scatter(packed)
    return partials.reshape(NC, BINS).sum(axis=0).reshape(2 * T, H, W)


def kernel(x, y, t, p):
    return _voxel_hist(x, y, t, p)
